# baseline (speedup 1.0000x reference)
import functools

import jax
import jax.numpy as jnp
from jax.experimental import pallas as pl
from jax.experimental.pallas import tpu as pltpu


def _se_body(x_ref, w1t_ref, w2t_ref, o_ref, *, inv_hw):
    xf = x_ref[...].astype(jnp.float32)
    pooled = jnp.sum(xf, axis=-1) * inv_hw
    h = jnp.dot(pooled, w1t_ref[...], preferred_element_type=jnp.float32)
    h = jnp.maximum(h, 0.0)
    s = jax.nn.sigmoid(
        jnp.dot(h, w2t_ref[...], preferred_element_type=jnp.float32))
    o_ref[...] = (xf * s[:, :, None]).astype(o_ref.dtype)


def _pick_tb(B, per_elem_bytes):
    tb = B
    for cand in range(B, 0, -1):
        if B % cand:
            continue
        steps = B // cand
        if steps % 2 == 0 and cand * per_elem_bytes <= (4 << 20):
            tb = cand
            break
    return tb


@jax.jit
def kernel(x_nchw, w1, w2):
    B, C, H, W = x_nchw.shape
    HW = H * W
    x = x_nchw.reshape(B, C, HW)
    w1t = w1.T
    w2t = w2.T

    per_elem_bytes = C * HW * x.dtype.itemsize
    TB = _pick_tb(B, per_elem_bytes)

    grid = (B // TB,)
    out = pl.pallas_call(
        functools.partial(_se_body, inv_hw=1.0 / float(HW)),
        out_shape=jax.ShapeDtypeStruct((B, C, HW), x.dtype),
        grid=grid,
        in_specs=[
            pl.BlockSpec((TB, C, HW), lambda b: (b, 0, 0)),
            pl.BlockSpec(w1t.shape, lambda b: (0, 0)),
            pl.BlockSpec(w2t.shape, lambda b: (0, 0)),
        ],
        out_specs=pl.BlockSpec((TB, C, HW), lambda b: (b, 0, 0)),
        compiler_params=pltpu.CompilerParams(
            dimension_semantics=("parallel",),
            vmem_limit_bytes=48 << 20,
        ),
        cost_estimate=pl.CostEstimate(
            flops=int(2 * B * C * HW + 4 * B * C * w1.shape[0]),
            transcendentals=int(B * C),
            bytes_accessed=int(2 * B * C * HW * x.dtype.itemsize),
        ),
    )(x, w1t, w2t)
    return out.reshape(B, C, H, W)
